# Initial kernel scaffold; baseline (speedup 1.0000x reference)
import jax
import jax.numpy as jnp
from jax.experimental import pallas as pl


def kernel(observations):
    raise NotImplementedError("write your pallas kernel here")


# speedup vs baseline: 2.3032x; 2.3032x over previous
import functools

import jax
import jax.numpy as jnp
from jax import lax
from jax.experimental import pallas as pl
from jax.experimental.pallas import tpu as pltpu
from jax.experimental.pallas import tpu_sc as plsc

NC = 2
NS = 16
NW = NC * NS
L = 16

B = 4096
T = 4096
W = 2 * T
NB = 64
VOCAB = 128
ROWS_PER_W = B // NW
INNER = W // (2 * L)


def _hist_body(obs_hbm, out_hbm, buf, hist, outb):
    c = lax.axis_index("c")
    s = lax.axis_index("s")
    wid = s * NC + c

    lanes = lax.iota(jnp.int32, L)
    lanebase = lanes * NB
    ones = jnp.full((L,), 1, jnp.int32)
    zeros = jnp.zeros((L,), jnp.int32)
    ihi0 = lanes * 2
    ilo0 = ihi0 + 1

    def row_body(r, _):
        row = wid * ROWS_PER_W + r
        pltpu.sync_copy(obs_hbm.at[row], buf)

        for j in range(NB):
            hist[pl.ds(j * L, L)] = zeros

        def inner(i, carry):
            ihi, ilo = carry
            hi = plsc.load_gather(buf, [ihi])
            lo = plsc.load_gather(buf, [ilo])
            addr = lanebase + (hi << 3) + lo
            plsc.addupdate_scatter(hist, [addr], ones)
            return (ihi + 2 * L, ilo + 2 * L)

        lax.fori_loop(0, INNER, inner, (ihi0, ilo0), unroll=4)

        for chunk in range(NB // L):
            acc = hist[pl.ds(chunk * L, L)]
            for lane in range(1, L):
                acc = acc + hist[pl.ds(lane * NB + chunk * L, L)]
            outb[pl.ds(chunk * L, L)] = acc
        for chunk in range(NB // L, VOCAB // L):
            outb[pl.ds(chunk * L, L)] = zeros

        pltpu.sync_copy(outb, out_hbm.at[row])
        return 0

    lax.fori_loop(0, ROWS_PER_W, row_body, 0)


@jax.jit
def kernel(observations):
    obs = observations.reshape(B, W)
    mesh = plsc.VectorSubcoreMesh(
        core_axis_name="c", subcore_axis_name="s", num_cores=NC, num_subcores=NS
    )
    run = pl.kernel(
        _hist_body,
        out_type=jax.ShapeDtypeStruct((B, VOCAB), jnp.int32),
        mesh=mesh,
        scratch_types=[
            pltpu.VMEM((W,), jnp.int32),
            pltpu.VMEM((NB * L,), jnp.int32),
            pltpu.VMEM((VOCAB,), jnp.int32),
        ],
        compiler_params=pltpu.CompilerParams(needs_layout_passes=False),
    )
    return run(obs)
